# baseline (speedup 1.0000x reference)
import jax
import jax.numpy as jnp
from jax import lax
from jax.experimental import pallas as pl
from jax.experimental.pallas import tpu as pltpu
from jax.experimental.pallas import tpu_sc as plsc

_M, _N = 16384, 256
_NW = 32
_ROWS_PER_W = _M // _NW
_CH_R = 32
_NCHUNK = _ROWS_PER_W // _CH_R
_NBUF = 4
_NJ = (_N // 2) // 16


def _sc_body(in_hbm, out_hbm, *refs):
    in_bufs = refs[:_NBUF]
    out_bufs = refs[_NBUF:2 * _NBUF]
    in_sems, out_sems = refs[2 * _NBUF], refs[2 * _NBUF + 1]
    wid = lax.axis_index("s") * 2 + lax.axis_index("c")
    row_base = wid * _ROWS_PER_W

    def in_copy(c, b):
        return pltpu.make_async_copy(
            in_hbm.at[pl.ds(row_base + c * _CH_R, _CH_R), :],
            in_bufs[b], in_sems.at[b])

    def out_copy(c, b):
        return pltpu.make_async_copy(
            out_bufs[b],
            out_hbm.at[pl.ds(row_base + c * _CH_R, _CH_R), :],
            out_sems.at[b])

    for b in range(_NBUF):
        in_copy(b, b).start()

    iota = lax.iota(jnp.int32, 16)
    idx_cols = [iota * 2 + (32 * jb) for jb in range(_NJ)]

    def super_chunk(i, _):
        for b in range(_NBUF):
            c = i * _NBUF + b
            in_copy(c, b).wait()

            @pl.when(c >= _NBUF)
            def _():
                out_copy(c - _NBUF, b).wait()

            in_buf = in_bufs[b]
            out_buf = out_bufs[b]

            @plsc.parallel_loop(0, _CH_R, unroll=4)
            def _row(r, in_buf=in_buf, out_buf=out_buf):
                idx_row = jnp.broadcast_to(r, (16,))
                for jb in range(_NJ):
                    v = plsc.load_gather(in_buf, [idx_row, idx_cols[jb]])
                    out_buf[r, pl.ds(16 * jb, 16)] = v

            out_copy(c, b).start()

            @pl.when(c + _NBUF < _NCHUNK)
            def _():
                in_copy(c + _NBUF, b).start()
        return _

    lax.fori_loop(0, _NCHUNK // _NBUF, super_chunk, None)
    for b in range(_NBUF):
        out_copy(_NCHUNK - _NBUF + b, b).wait()


def kernel(inputs):
    k = pl.kernel(
        _sc_body,
        out_type=jax.ShapeDtypeStruct((_M, _N // 2), jnp.float32),
        mesh=plsc.VectorSubcoreMesh(core_axis_name="c", subcore_axis_name="s"),
        compiler_params=pltpu.CompilerParams(
            needs_layout_passes=False, skip_device_barrier=True),
        scratch_types=(
            [pltpu.VMEM((_CH_R, _N), jnp.float32) for _ in range(_NBUF)]
            + [pltpu.VMEM((_CH_R, _N // 2), jnp.float32) for _ in range(_NBUF)]
            + [pltpu.SemaphoreType.DMA((_NBUF,)),
               pltpu.SemaphoreType.DMA((_NBUF,))]
        ),
    )
    return k(inputs)


# speedup vs baseline: 1.0096x; 1.0096x over previous
import jax
import jax.numpy as jnp
from jax import lax
from jax.experimental import pallas as pl
from jax.experimental.pallas import tpu as pltpu
from jax.experimental.pallas import tpu_sc as plsc

_M, _N = 16384, 256
_NW = 32
_ROWS_PER_W = _M // _NW
_CH_R = 32
_NCHUNK = _ROWS_PER_W // _CH_R
_NBUF = 4
_NJ = (_N // 2) // 16


def _sc_body(in_hbm, out_hbm, *refs):
    in_bufs = refs[:_NBUF]
    out_bufs = refs[_NBUF:2 * _NBUF]
    in_sems, out_sems = refs[2 * _NBUF], refs[2 * _NBUF + 1]
    wid = lax.axis_index("s") * 2 + lax.axis_index("c")
    row_base = wid * _ROWS_PER_W

    _H = _CH_R // 2

    def in_copies(c, b):
        r0 = row_base + c * _CH_R
        return (
            pltpu.make_async_copy(
                in_hbm.at[pl.ds(r0, _H), :],
                in_bufs[b].at[pl.ds(0, _H), :], in_sems.at[b, 0]),
            pltpu.make_async_copy(
                in_hbm.at[pl.ds(r0 + _H, _H), :],
                in_bufs[b].at[pl.ds(_H, _H), :], in_sems.at[b, 1]),
        )

    def in_start(c, b):
        for cp in in_copies(c, b):
            cp.start()

    def in_wait(c, b):
        for cp in in_copies(c, b):
            cp.wait()

    def out_copy(c, b):
        return pltpu.make_async_copy(
            out_bufs[b],
            out_hbm.at[pl.ds(row_base + c * _CH_R, _CH_R), :],
            out_sems.at[b])

    for b in range(_NBUF):
        in_start(b, b)

    iota = lax.iota(jnp.int32, 16)
    idx_cols = [iota * 2 + (32 * jb) for jb in range(_NJ)]

    def super_chunk(i, _):
        for b in range(_NBUF):
            c = i * _NBUF + b
            in_wait(c, b)

            @pl.when(c >= _NBUF)
            def _():
                out_copy(c - _NBUF, b).wait()

            in_buf = in_bufs[b]
            out_buf = out_bufs[b]

            @plsc.parallel_loop(0, _CH_R, unroll=4)
            def _row(r, in_buf=in_buf, out_buf=out_buf):
                idx_row = jnp.broadcast_to(r, (16,))
                for jb in range(_NJ):
                    v = plsc.load_gather(in_buf, [idx_row, idx_cols[jb]])
                    out_buf[r, pl.ds(16 * jb, 16)] = v

            out_copy(c, b).start()

            @pl.when(c + _NBUF < _NCHUNK)
            def _():
                in_start(c + _NBUF, b)
        return _

    lax.fori_loop(0, _NCHUNK // _NBUF, super_chunk, None)
    for b in range(_NBUF):
        out_copy(_NCHUNK - _NBUF + b, b).wait()


def kernel(inputs):
    k = pl.kernel(
        _sc_body,
        out_type=jax.ShapeDtypeStruct((_M, _N // 2), jnp.float32),
        mesh=plsc.VectorSubcoreMesh(core_axis_name="c", subcore_axis_name="s"),
        compiler_params=pltpu.CompilerParams(
            needs_layout_passes=False, skip_device_barrier=True),
        scratch_types=(
            [pltpu.VMEM((_CH_R, _N), jnp.float32) for _ in range(_NBUF)]
            + [pltpu.VMEM((_CH_R, _N // 2), jnp.float32) for _ in range(_NBUF)]
            + [pltpu.SemaphoreType.DMA((_NBUF, 2)),
               pltpu.SemaphoreType.DMA((_NBUF,))]
        ),
    )
    return k(inputs)


# speedup vs baseline: 1.0105x; 1.0009x over previous
import jax
import jax.numpy as jnp
from jax import lax
from jax.experimental import pallas as pl
from jax.experimental.pallas import tpu as pltpu
from jax.experimental.pallas import tpu_sc as plsc

_M, _N = 16384, 256
_NW = 32
_ROWS_PER_W = _M // _NW
_CH_R = 32
_NCHUNK = _ROWS_PER_W // _CH_R
_NBUF = 4
_NJ = (_N // 2) // 16


def _sc_body(in_hbm, out_hbm, *refs):
    in_bufs = refs[:_NBUF]
    out_bufs = refs[_NBUF:2 * _NBUF]
    in_sems, out_sems = refs[2 * _NBUF], refs[2 * _NBUF + 1]
    wid = lax.axis_index("s") * 2 + lax.axis_index("c")
    row_base = wid * _ROWS_PER_W

    _H = _CH_R // 2

    def in_copies(c, b):
        r0 = row_base + c * _CH_R
        return (
            pltpu.make_async_copy(
                in_hbm.at[pl.ds(r0, _H), :],
                in_bufs[b].at[pl.ds(0, _H), :], in_sems.at[b, 0]),
            pltpu.make_async_copy(
                in_hbm.at[pl.ds(r0 + _H, _H), :],
                in_bufs[b].at[pl.ds(_H, _H), :], in_sems.at[b, 1]),
        )

    def in_start(c, b):
        for cp in in_copies(c, b):
            cp.start()

    def in_wait(c, b):
        for cp in in_copies(c, b):
            cp.wait()

    def out_copies(c, b):
        r0 = row_base + c * _CH_R
        return (
            pltpu.make_async_copy(
                out_bufs[b].at[pl.ds(0, _H), :],
                out_hbm.at[pl.ds(r0, _H), :], out_sems.at[b, 0]),
            pltpu.make_async_copy(
                out_bufs[b].at[pl.ds(_H, _H), :],
                out_hbm.at[pl.ds(r0 + _H, _H), :], out_sems.at[b, 1]),
        )

    def out_start(c, b):
        for cp in out_copies(c, b):
            cp.start()

    def out_wait(c, b):
        for cp in out_copies(c, b):
            cp.wait()

    for b in range(_NBUF):
        in_start(b, b)

    iota = lax.iota(jnp.int32, 16)
    idx_cols = [iota * 2 + (32 * jb) for jb in range(_NJ)]

    def super_chunk(i, _):
        for b in range(_NBUF):
            c = i * _NBUF + b
            in_wait(c, b)

            @pl.when(c >= _NBUF)
            def _():
                out_wait(c - _NBUF, b)

            in_buf = in_bufs[b]
            out_buf = out_bufs[b]

            @plsc.parallel_loop(0, _CH_R, unroll=4)
            def _row(r, in_buf=in_buf, out_buf=out_buf):
                idx_row = jnp.broadcast_to(r, (16,))
                for jb in range(_NJ):
                    v = plsc.load_gather(in_buf, [idx_row, idx_cols[jb]])
                    out_buf[r, pl.ds(16 * jb, 16)] = v

            out_start(c, b)

            @pl.when(c + _NBUF < _NCHUNK)
            def _():
                in_start(c + _NBUF, b)
        return _

    lax.fori_loop(0, _NCHUNK // _NBUF, super_chunk, None)
    for b in range(_NBUF):
        out_wait(_NCHUNK - _NBUF + b, b)


def kernel(inputs):
    k = pl.kernel(
        _sc_body,
        out_type=jax.ShapeDtypeStruct((_M, _N // 2), jnp.float32),
        mesh=plsc.VectorSubcoreMesh(core_axis_name="c", subcore_axis_name="s"),
        compiler_params=pltpu.CompilerParams(
            needs_layout_passes=False, skip_device_barrier=True),
        scratch_types=(
            [pltpu.VMEM((_CH_R, _N), jnp.float32) for _ in range(_NBUF)]
            + [pltpu.VMEM((_CH_R, _N // 2), jnp.float32) for _ in range(_NBUF)]
            + [pltpu.SemaphoreType.DMA((_NBUF, 2)),
               pltpu.SemaphoreType.DMA((_NBUF, 2))]
        ),
    )
    return k(inputs)
